# baseline (speedup 1.0000x reference)
import jax
import jax.numpy as jnp
from jax.experimental import pallas as pl

_B = 16384


def _loss_kernel(logits_ref, targets_ref, out_ref):
    i = pl.program_id(0)

    @pl.when(i == 0)
    def _init():
        out_ref[...] = jnp.zeros_like(out_ref)

    l = logits_ref[...]
    lt = jnp.swapaxes(l, 0, 1)
    t = targets_ref[0]
    x = jnp.sign(t) * jnp.log(jnp.abs(t) + 1.0) + 20.0
    row = jax.lax.broadcasted_iota(jnp.int32, (lt.shape[0], 1), 0).astype(jnp.float32)
    sel = jnp.maximum(1.0 - jnp.abs(x - row), 0.0)
    s1 = jnp.sum(jnp.exp(lt), axis=0, keepdims=True)
    s2 = jnp.sum(sel * lt, axis=0, keepdims=True)
    loss = jnp.log(s1) - s2
    out_ref[...] += jnp.sum(loss, axis=1, keepdims=True)


def kernel(logits, targets):
    n, nb = logits.shape
    t3 = targets.reshape(n // _B, 1, _B)
    out = pl.pallas_call(
        _loss_kernel,
        grid=(n // _B,),
        in_specs=[
            pl.BlockSpec((_B, nb), lambda i: (i, 0)),
            pl.BlockSpec((1, 1, _B), lambda i: (i, 0, 0)),
        ],
        out_specs=pl.BlockSpec((1, 1), lambda i: (0, 0)),
        out_shape=jax.ShapeDtypeStruct((1, 1), jnp.float32),
    )(logits, t3)
    return (out[0, 0] / n).astype(jnp.float32)


# speedup vs baseline: 1.0165x; 1.0165x over previous
import jax
import jax.numpy as jnp
from jax.experimental import pallas as pl

_B = 32768


def _loss_kernel(logits_ref, targets_ref, out_ref):
    i = pl.program_id(0)

    @pl.when(i == 0)
    def _init():
        out_ref[...] = jnp.zeros_like(out_ref)

    l = logits_ref[...]
    lt = jnp.swapaxes(l, 0, 1)
    t = targets_ref[0]
    x = jnp.sign(t) * jnp.log(jnp.abs(t) + 1.0) + 20.0
    row = jax.lax.broadcasted_iota(jnp.int32, (lt.shape[0], 1), 0).astype(jnp.float32)
    sel = jnp.maximum(1.0 - jnp.abs(x - row), 0.0)
    s1 = jnp.sum(jnp.exp(lt), axis=0, keepdims=True)
    s2 = jnp.sum(sel * lt, axis=0, keepdims=True)
    loss = jnp.log(s1) - s2
    out_ref[...] += jnp.sum(loss, axis=1, keepdims=True)


def kernel(logits, targets):
    n, nb = logits.shape
    t3 = targets.reshape(n // _B, 1, _B)
    out = pl.pallas_call(
        _loss_kernel,
        grid=(n // _B,),
        in_specs=[
            pl.BlockSpec((_B, nb), lambda i: (i, 0)),
            pl.BlockSpec((1, 1, _B), lambda i: (i, 0, 0)),
        ],
        out_specs=pl.BlockSpec((1, 1), lambda i: (0, 0)),
        out_shape=jax.ShapeDtypeStruct((1, 1), jnp.float32),
    )(logits, t3)
    return (out[0, 0] / n).astype(jnp.float32)
